# baseline (speedup 1.0000x reference)
import functools

import jax
import jax.numpy as jnp
from jax import lax
from jax.experimental import pallas as pl
from jax.experimental.pallas import tpu as pltpu
from jax.experimental.pallas import tpu_sc as plsc

BATCH = 16384
DIM = 100

_INFO = plsc.get_sparse_core_info()
_NC = _INFO.num_cores
_NS = _INFO.num_subcores
NW = _NC * _NS
B_PER_W = BATCH // NW
CHUNK = 128
N_CHUNKS = B_PER_W // CHUNK


@functools.partial(
    pl.kernel,
    mesh=plsc.VectorSubcoreMesh(core_axis_name="c", subcore_axis_name="s"),
    out_type=jax.ShapeDtypeStruct((BATCH, DIM), jnp.float32),
    scratch_types=[
        pltpu.VMEM((N_CHUNKS, CHUNK), jnp.int32),
        pltpu.VMEM((B_PER_W, DIM), jnp.float32),
        pltpu.SemaphoreType.DMA,
    ],
    compiler_params=pltpu.CompilerParams(use_tc_tiling_on_sc=False),
)
def _emb_gather(idx_hbm, table_hbm, out_hbm, idx_v, rows_v, sem):
    wid = lax.axis_index("s") * _NC + lax.axis_index("c")
    base = wid * B_PER_W
    pltpu.sync_copy(idx_hbm.at[wid], idx_v)
    copies = [
        pltpu.async_copy(
            table_hbm.at[idx_v.at[j]],
            rows_v.at[pl.ds(j * CHUNK, CHUNK)],
            sem,
        )
        for j in range(N_CHUNKS)
    ]
    for c in copies:
        c.wait()
    pltpu.sync_copy(rows_v, out_hbm.at[pl.ds(base, B_PER_W)])


def kernel(inputs, table):
    idx = inputs.reshape(NW, N_CHUNKS, CHUNK)
    return _emb_gather(idx, table)


# speedup vs baseline: 5.8106x; 5.8106x over previous
import functools

import jax
import jax.numpy as jnp
from jax import lax
from jax.experimental import pallas as pl
from jax.experimental.pallas import tpu as pltpu
from jax.experimental.pallas import tpu_sc as plsc

BATCH = 16384
DIM = 100

_INFO = plsc.get_sparse_core_info()
_NC = _INFO.num_cores
_NS = _INFO.num_subcores
NW = _NC * _NS
B_PER_W = BATCH // NW
FLIGHT = 16
N_FLIGHTS = B_PER_W // FLIGHT


@functools.partial(
    pl.kernel,
    mesh=plsc.VectorSubcoreMesh(core_axis_name="c", subcore_axis_name="s"),
    out_type=jax.ShapeDtypeStruct((BATCH, DIM), jnp.float32),
    scratch_types=[
        pltpu.VMEM((B_PER_W,), jnp.int32),
        pltpu.VMEM((B_PER_W, DIM), jnp.float32),
        pltpu.SemaphoreType.DMA,
    ],
    compiler_params=pltpu.CompilerParams(use_tc_tiling_on_sc=True),
)
def _emb_gather(idx_hbm, table_hbm, out_hbm, idx_v, rows_v, sem):
    wid = lax.axis_index("s") * _NC + lax.axis_index("c")
    base = wid * B_PER_W
    pltpu.sync_copy(idx_hbm.at[pl.ds(base, B_PER_W)], idx_v)

    def fire(g):
        i0 = g * FLIGHT
        vec = idx_v[pl.ds(i0, FLIGHT)]
        for k in range(FLIGHT):
            pltpu.async_copy(
                table_hbm.at[pl.ds(vec[k], 1)],
                rows_v.at[pl.ds(i0 + k, 1)],
                sem,
            )

    def drain(g):
        i0 = g * FLIGHT
        for k in range(FLIGHT):
            pltpu.make_async_copy(
                table_hbm.at[pl.ds(0, 1)],
                rows_v.at[pl.ds(i0 + k, 1)],
                sem,
            ).wait()

    def body(g, _):
        fire(g)

        @pl.when(g > 0)
        def _():
            drain(g - 1)

        return ()

    lax.fori_loop(0, N_FLIGHTS, body, (), unroll=False)
    drain(N_FLIGHTS - 1)
    pltpu.sync_copy(rows_v, out_hbm.at[pl.ds(base, B_PER_W)])


def kernel(inputs, table):
    idx = inputs.reshape(BATCH)
    return _emb_gather(idx, table)


# speedup vs baseline: 7.2961x; 1.2556x over previous
import functools

import jax
import jax.numpy as jnp
import numpy as np
from jax import lax
from jax.experimental import pallas as pl
from jax.experimental.pallas import tpu as pltpu
from jax.experimental.pallas import tpu_sc as plsc

BATCH = 16384
DIM = 100
ROW_PAD = 104

_INFO = plsc.get_sparse_core_info()
_NC = _INFO.num_cores
_NS = _INFO.num_subcores
NW = _NC * _NS
LANES = 16

VOCAB_PAD = 1000064
N_TILE_COLS = VOCAB_PAD // 128
STRIPS_BASE = N_TILE_COLS // NW
STRIPS_REM = N_TILE_COLS % NW

HIT_CAP = 768
N_IDX_CHUNKS = BATCH // LANES
SLOTS = 16
SLOT_STRIDE = 112


@functools.partial(
    pl.kernel,
    mesh=plsc.VectorSubcoreMesh(core_axis_name="c", subcore_axis_name="s"),
    out_type=jax.ShapeDtypeStruct((BATCH * ROW_PAD,), jnp.float32),
    scratch_types=[
        pltpu.VMEM((BATCH,), jnp.int32),
        pltpu.VMEM((100, 128), jnp.float32),
        pltpu.VMEM((100, 128), jnp.float32),
        pltpu.VMEM((HIT_CAP + LANES,), jnp.int32),
        pltpu.VMEM((HIT_CAP + LANES,), jnp.int32),
        pltpu.VMEM((HIT_CAP + LANES,), jnp.int32),
        pltpu.VMEM((HIT_CAP + LANES,), jnp.int32),
        pltpu.VMEM((SLOTS * SLOT_STRIDE + LANES,), jnp.float32),
        pltpu.SemaphoreType.DMA,
        pltpu.SemaphoreType.DMA,
    ],
    compiler_params=pltpu.CompilerParams(
        use_tc_tiling_on_sc=True, needs_layout_passes=False),
)
def _emb_stream(idx_hbm, table_t_hbm, out_hbm, idx_all, strip_a, strip_b,
                hit_r, hit_i, s_r, s_i, outbuf, ssem, osem):
    wid = lax.axis_index("s") * _NC + lax.axis_index("c")
    lo_strip = wid * STRIPS_BASE + jnp.minimum(wid, STRIPS_REM)
    n_strips = STRIPS_BASE + jnp.where(wid < STRIPS_REM, 1, 0)
    lo_c = lo_strip
    hi_c = lo_strip + n_strips

    pltpu.sync_copy(idx_hbm, idx_all)

    iota = lax.iota(jnp.int32, LANES)
    minus1 = jnp.full((LANES,), -1, jnp.int32)

    def prefill(n, _):
        hit_r[pl.ds(n * LANES, LANES)] = minus1
        return ()
    lax.fori_loop(0, (HIT_CAP + LANES) // LANES, prefill, (), unroll=False)

    def scan(n, nh):
        v = idx_all[pl.ds(n * LANES, LANES)]
        c = lax.shift_right_logical(v, 7)
        m = (c >= jnp.full((LANES,), lo_c, jnp.int32)) & (
            c < jnp.full((LANES,), hi_c, jnp.int32))
        cnt = plsc.cumsum(jnp.where(m, jnp.full((LANES,), 1, jnp.int32), jnp.full((LANES,), 0, jnp.int32)))
        dest = jnp.full((LANES,), nh, jnp.int32) + cnt - 1
        plsc.store_scatter(hit_r, [dest], v, mask=m)
        pos = jnp.full((LANES,), n * LANES, jnp.int32) + iota
        plsc.store_scatter(hit_i, [dest], pos, mask=m)
        return nh + cnt[LANES - 1]
    nh = lax.fori_loop(0, N_IDX_CHUNKS, scan, jnp.int32(0), unroll=False)
    nh_chunks = lax.shift_right_logical(nh + LANES - 1, 4)

    def fetch_strip(g, buf):
        col0 = pl.multiple_of((lo_strip + g) * 128, 128)
        return pltpu.async_copy(table_t_hbm.at[:, pl.ds(col0, 128)], buf, ssem)

    fetch_strip(0, strip_a)

    def extract_hits(g, j0, scnt, buf):
        def one(k, j):
            rv = s_r[pl.ds(k, LANES)]
            iv = s_i[pl.ds(k, LANES)]
            r = rv[0]
            i_out = iv[0]
            lane = lax.rem(r, jnp.int32(128))
            colv = jnp.full((LANES,), lane, jnp.int32)
            slot = lax.rem(j, jnp.int32(SLOTS))
            sbase = slot * SLOT_STRIDE
            for t in range(7):
                rowv = iota + t * LANES
                if t == 6:
                    rowv = jnp.minimum(rowv, DIM - 1)
                g16 = plsc.load_gather(buf, [rowv, colv])
                outbuf[pl.ds(sbase + t * LANES, LANES)] = g16

            @pl.when(j >= SLOTS)
            def _():
                pltpu.make_async_copy(
                    out_hbm.at[pl.ds(0, ROW_PAD)],
                    outbuf.at[pl.ds(sbase, ROW_PAD)],
                    osem,
                ).wait()

            pltpu.async_copy(
                outbuf.at[pl.ds(sbase, ROW_PAD)],
                out_hbm.at[pl.ds(i_out * ROW_PAD, ROW_PAD)],
                osem,
            )
            return j + 1
        return lax.fori_loop(0, scnt, one, j0, unroll=False)

    def strip_body(g, j0):
        c_this = lo_strip + g

        def rescan(n, scnt):
            rv = hit_r[pl.ds(n * LANES, LANES)]
            iv = hit_i[pl.ds(n * LANES, LANES)]
            m = lax.shift_right_logical(rv, 7) == jnp.full(
                (LANES,), c_this, jnp.int32)
            cnt = plsc.cumsum(jnp.where(m, jnp.full((LANES,), 1, jnp.int32), jnp.full((LANES,), 0, jnp.int32)))
            dest = jnp.full((LANES,), scnt, jnp.int32) + cnt - 1
            plsc.store_scatter(s_r, [dest], rv, mask=m)
            plsc.store_scatter(s_i, [dest], iv, mask=m)
            return scnt + cnt[LANES - 1]
        scnt = lax.fori_loop(0, nh_chunks, rescan, jnp.int32(0), unroll=False)

        parity = lax.rem(g, jnp.int32(2))

        def wait_strip(buf):
            pltpu.make_async_copy(
                table_t_hbm.at[:, pl.ds(0, 128)], buf, ssem).wait()

        j_box = [j0]

        @pl.when(parity == 0)
        def _():
            wait_strip(strip_a)

            @pl.when(g + 1 < n_strips)
            def _():
                fetch_strip(g + 1, strip_b)

        @pl.when(parity == 1)
        def _():
            wait_strip(strip_b)

            @pl.when(g + 1 < n_strips)
            def _():
                fetch_strip(g + 1, strip_a)

        ja = extract_hits(g, j0, jnp.where(parity == 0, scnt, 0), strip_a)
        jb = extract_hits(g, ja, jnp.where(parity == 1, scnt, 0), strip_b)
        return jb

    j_total = lax.fori_loop(0, n_strips, strip_body, jnp.int32(0),
                            unroll=False)

    def final_drain(_, __):
        pltpu.make_async_copy(
            out_hbm.at[pl.ds(0, ROW_PAD)],
            outbuf.at[pl.ds(0, ROW_PAD)],
            osem,
        ).wait()
        return ()
    lax.fori_loop(0, jnp.minimum(j_total, SLOTS), final_drain, (),
                  unroll=False)


def kernel(inputs, table):
    idx = inputs.reshape(BATCH)
    out_pad = _emb_stream(idx, table.T)
    return out_pad.reshape(BATCH, ROW_PAD)[:, :DIM]
